# baseline (speedup 1.0000x reference)
import functools

import jax
import jax.numpy as jnp
from jax import lax
from jax.experimental import pallas as pl
from jax.experimental.pallas import tpu as pltpu
from jax.experimental.pallas import tpu_sc as plsc

_NUM_CORES = 2
_NUM_SUBCORES = 16
_NW = _NUM_CORES * _NUM_SUBCORES
_CHUNK = 128
_DP = 128
_NBUF = 4


def _gather_call(idx, table_pad, n_ch):
    N = _NW * n_ch * _CHUNK
    mesh = plsc.VectorSubcoreMesh(core_axis_name="c", subcore_axis_name="s")

    @functools.partial(
        pl.kernel,
        mesh=mesh,
        out_type=jax.ShapeDtypeStruct((N, _DP), jnp.float32),
        scratch_types=[
            pltpu.VMEM((n_ch, _CHUNK), jnp.int32),
            pltpu.VMEM((_NBUF, _CHUNK, _DP), jnp.float32),
            pltpu.SemaphoreType.DMA((_NBUF,)),
            pltpu.SemaphoreType.DMA((_NBUF,)),
        ],
    )
    def run(idx_hbm, table_hbm, out_hbm, idx_v, rows_v, gsem, wsem):
        wid = lax.axis_index("s") * _NUM_CORES + lax.axis_index("c")
        base = wid * (n_ch * _CHUNK)
        pltpu.sync_copy(idx_hbm.at[wid], idx_v)

        def gather(j, b):
            pltpu.async_copy(table_hbm.at[idx_v.at[j]], rows_v.at[b], gsem.at[b])

        def gather_wait(j, b):
            pltpu.make_async_copy(
                table_hbm.at[idx_v.at[j]], rows_v.at[b], gsem.at[b]
            ).wait()

        def write(j, b):
            pltpu.async_copy(
                rows_v.at[b],
                out_hbm.at[pl.ds(base + j * _CHUNK, _CHUNK)],
                wsem.at[b],
            )

        def write_wait(j, b):
            pltpu.make_async_copy(
                rows_v.at[b],
                out_hbm.at[pl.ds(base + j * _CHUNK, _CHUNK)],
                wsem.at[b],
            ).wait()

        for b in range(_NBUF):
            gather(b, b)

        def body(g, carry):
            j_prev = (g - 1) * _NBUF
            j_next = g * _NBUF
            for b in range(_NBUF):
                gather_wait(j_prev + b, b)
                write(j_prev + b, b)
            for b in range(_NBUF):
                write_wait(j_prev + b, b)
                gather(j_next + b, b)
            return carry

        lax.fori_loop(1, n_ch // _NBUF, body, 0)

        j_last = n_ch - _NBUF
        for b in range(_NBUF):
            gather_wait(j_last + b, b)
            write(j_last + b, b)
        for b in range(_NBUF):
            write_wait(j_last + b, b)

    return run(idx, table_pad)


def _pad_tc(table, D):
    V = table.shape[0]
    BR = 1024

    def body(x_ref, o_ref):
        o_ref[:, :D] = x_ref[...]
        o_ref[:, D:] = jnp.zeros((BR, _DP - D), jnp.float32)

    return pl.pallas_call(
        body,
        grid=(pl.cdiv(V, BR),),
        in_specs=[pl.BlockSpec((BR, D), lambda i: (i, 0))],
        out_specs=pl.BlockSpec((BR, _DP), lambda i: (i, 0)),
        out_shape=jax.ShapeDtypeStruct((V, _DP), jnp.float32),
    )(table)


def _slice_tc(x3, D):
    B, S, _ = x3.shape
    BB = 16

    def body(x_ref, o_ref):
        o_ref[...] = x_ref[:, :, :D]

    return pl.pallas_call(
        body,
        grid=(B // BB,),
        in_specs=[pl.BlockSpec((BB, S, _DP), lambda i: (i, 0, 0))],
        out_specs=pl.BlockSpec((BB, S, D), lambda i: (i, 0, 0)),
        out_shape=jax.ShapeDtypeStruct((B, S, D), jnp.float32),
    )(x3)


def kernel(doc_batch, table):
    B, S = doc_batch.shape
    V, D = table.shape
    N = B * S
    n_ch = N // (_NW * _CHUNK)
    idx = doc_batch.astype(jnp.int32).reshape(_NW, n_ch, _CHUNK)
    table_pad = _pad_tc(table, D)
    out = _gather_call(idx, table_pad, n_ch)
    return _slice_tc(out.reshape(B, S, _DP), D)


# speedup vs baseline: 1.4042x; 1.4042x over previous
import functools

import jax
import jax.numpy as jnp
from jax import lax
from jax.experimental import pallas as pl
from jax.experimental.pallas import tpu as pltpu
from jax.experimental.pallas import tpu_sc as plsc

_NUM_CORES = 2
_NUM_SUBCORES = 16
_NW = _NUM_CORES * _NUM_SUBCORES
_CHUNK = 128
_DP = 128
_NBUF = 5


def _gather_call(idx, table_pad, n_ch):
    N = _NW * n_ch * _CHUNK
    mesh = plsc.VectorSubcoreMesh(core_axis_name="c", subcore_axis_name="s")

    @functools.partial(
        pl.kernel,
        mesh=mesh,
        out_type=jax.ShapeDtypeStruct((N, _DP), jnp.float32),
        scratch_types=[
            pltpu.VMEM((n_ch, _CHUNK), jnp.int32),
            pltpu.VMEM((_NBUF, _CHUNK, _DP), jnp.float32),
            pltpu.SemaphoreType.DMA((_NBUF,)),
            pltpu.SemaphoreType.DMA((_NBUF,)),
        ],
    )
    def run(idx_hbm, table_hbm, out_hbm, idx_v, rows_v, gsem, wsem):
        wid = lax.axis_index("s") * _NUM_CORES + lax.axis_index("c")
        base = wid * (n_ch * _CHUNK)
        pltpu.sync_copy(idx_hbm.at[wid], idx_v)

        def gather(j, b):
            pltpu.async_copy(table_hbm.at[idx_v.at[j]], rows_v.at[b], gsem.at[b])

        def gather_wait(j, b):
            pltpu.make_async_copy(
                table_hbm.at[idx_v.at[j]], rows_v.at[b], gsem.at[b]
            ).wait()

        def write(j, b):
            pltpu.async_copy(
                rows_v.at[b],
                out_hbm.at[pl.ds(base + j * _CHUNK, _CHUNK)],
                wsem.at[b],
            )

        def write_wait(j, b):
            pltpu.make_async_copy(
                rows_v.at[b],
                out_hbm.at[pl.ds(base + j * _CHUNK, _CHUNK)],
                wsem.at[b],
            ).wait()

        for b in range(_NBUF):
            gather(b, b)

        def body(g, carry):
            j_prev = (g - 1) * _NBUF
            j_next = g * _NBUF
            for b in range(_NBUF):
                gather_wait(j_prev + b, b)
                write(j_prev + b, b)
            for b in range(_NBUF):
                write_wait(j_prev + b, b)
                gather(j_next + b, b)
            return carry

        lax.fori_loop(1, n_ch // _NBUF, body, 0)

        j_last = n_ch - _NBUF
        for b in range(_NBUF):
            gather_wait(j_last + b, b)
            write(j_last + b, b)
        for b in range(_NBUF):
            write_wait(j_last + b, b)

    return run(idx, table_pad)


def kernel(doc_batch, table):
    B, S = doc_batch.shape
    V, D = table.shape
    N = B * S
    n_ch = N // (_NW * _CHUNK)
    idx = doc_batch.astype(jnp.int32).reshape(_NW, n_ch, _CHUNK)
    table_pad = jnp.pad(table, ((0, 0), (0, _DP - D)))
    out = _gather_call(idx, table_pad, n_ch)
    return out.reshape(B, S, _DP)[:, :, :D]


# speedup vs baseline: 1.4053x; 1.0008x over previous
import functools

import jax
import jax.numpy as jnp
from jax import lax
from jax.experimental import pallas as pl
from jax.experimental.pallas import tpu as pltpu
from jax.experimental.pallas import tpu_sc as plsc

_NUM_CORES = 2
_NUM_SUBCORES = 16
_NW = _NUM_CORES * _NUM_SUBCORES
_CHUNK = 128
_DP = 128
_NBUF = 5


def _gather_call(idx, table_pad, n_ch):
    N = _NW * n_ch * _CHUNK
    mesh = plsc.VectorSubcoreMesh(core_axis_name="c", subcore_axis_name="s")

    @functools.partial(
        pl.kernel,
        mesh=mesh,
        out_type=jax.ShapeDtypeStruct((N, _DP), jnp.float32),
        scratch_types=[
            pltpu.VMEM((n_ch, _CHUNK), jnp.int32),
            pltpu.VMEM((_NBUF, _CHUNK, _DP), jnp.float32),
            pltpu.SemaphoreType.DMA((_NBUF,)),
            pltpu.SemaphoreType.DMA((_NBUF,)),
        ],
    )
    def run(idx_hbm, table_hbm, out_hbm, idx_v, rows_v, gsem, wsem):
        wid = lax.axis_index("s") * _NUM_CORES + lax.axis_index("c")
        base = wid * (n_ch * _CHUNK)
        pltpu.sync_copy(idx_hbm.at[wid], idx_v)

        def gather(j, b):
            pltpu.async_copy(table_hbm.at[idx_v.at[j]], rows_v.at[b], gsem.at[b])

        def gather_wait(j, b):
            pltpu.make_async_copy(
                table_hbm.at[idx_v.at[j]], rows_v.at[b], gsem.at[b]
            ).wait()

        def write(j, b):
            pltpu.async_copy(
                rows_v.at[b],
                out_hbm.at[pl.ds(base + j * _CHUNK, _CHUNK)],
                wsem.at[b],
            )

        def write_wait(j, b):
            pltpu.make_async_copy(
                rows_v.at[b],
                out_hbm.at[pl.ds(base + j * _CHUNK, _CHUNK)],
                wsem.at[b],
            ).wait()

        for b in range(_NBUF):
            gather(b, b)

        def body(g, carry):
            j_prev = (g - 1) * _NBUF
            j_next = g * _NBUF
            for b in range(_NBUF):
                gather_wait(j_prev + b, b)
                write(j_prev + b, b)
            for b in range(_NBUF):
                write_wait(j_prev + b, b)
                gather(j_next + b, b)
            return carry

        lax.fori_loop(1, n_ch // _NBUF, body, 0)

        j_last = n_ch - _NBUF
        for b in range(_NBUF):
            gather_wait(j_last + b, b)
            write(j_last + b, b)
        for b in range(_NBUF):
            write_wait(j_last + b, b)

    return run(idx, table_pad)


def kernel(doc_batch, table):
    B, S = doc_batch.shape
    V, D = table.shape
    N = B * S
    n_ch = N // (_NW * _CHUNK)
    idx = doc_batch.astype(jnp.int32).reshape(_NW, n_ch, _CHUNK)
    vpad = (-V) % 8
    table_pad = jnp.pad(table, ((0, vpad), (0, _DP - D)))
    out = _gather_call(idx, table_pad, n_ch)
    return out.reshape(B, S, _DP)[:, :, :D]


# speedup vs baseline: 1.7064x; 1.2143x over previous
import functools

import jax
import jax.numpy as jnp
from jax import lax
from jax.experimental import pallas as pl
from jax.experimental.pallas import tpu as pltpu
from jax.experimental.pallas import tpu_sc as plsc

_NUM_CORES = 2
_NUM_SUBCORES = 16
_NW = _NUM_CORES * _NUM_SUBCORES
_CHUNK = 128
_DP = 128
_NBUF = 5


def _gather_call(idx, table_pad, n_ch):
    N = _NW * n_ch * _CHUNK
    mesh = plsc.VectorSubcoreMesh(core_axis_name="c", subcore_axis_name="s")

    @functools.partial(
        pl.kernel,
        mesh=mesh,
        out_type=jax.ShapeDtypeStruct((N, _DP), jnp.float32),
        scratch_types=[
            pltpu.VMEM((n_ch, _CHUNK), jnp.int32),
            pltpu.VMEM((_NBUF, _CHUNK, _DP), jnp.float32),
            pltpu.SemaphoreType.DMA((_NBUF,)),
            pltpu.SemaphoreType.DMA((_NBUF,)),
        ],
    )
    def run(idx_hbm, table_hbm, out_hbm, idx_v, rows_v, gsem, wsem):
        wid = lax.axis_index("s") * _NUM_CORES + lax.axis_index("c")
        base = wid * (n_ch * _CHUNK)
        pltpu.sync_copy(idx_hbm.at[wid], idx_v)

        def gather(j, b):
            pltpu.async_copy(table_hbm.at[idx_v.at[j]], rows_v.at[b], gsem.at[b])

        def gather_wait(j, b):
            pltpu.make_async_copy(
                table_hbm.at[idx_v.at[j]], rows_v.at[b], gsem.at[b]
            ).wait()

        def write(j, b):
            pltpu.async_copy(
                rows_v.at[b],
                out_hbm.at[pl.ds(base + j * _CHUNK, _CHUNK)],
                wsem.at[b],
            )

        def write_wait(j, b):
            pltpu.make_async_copy(
                rows_v.at[b],
                out_hbm.at[pl.ds(base + j * _CHUNK, _CHUNK)],
                wsem.at[b],
            ).wait()

        for b in range(_NBUF):
            gather(b, b)

        def body(g, carry):
            j_prev = (g - 1) * _NBUF
            j_next = g * _NBUF
            for b in range(_NBUF):
                gather_wait(j_prev + b, b)
                write(j_prev + b, b)
            for b in range(_NBUF):
                write_wait(j_prev + b, b)
                gather(j_next + b, b)
            return carry

        lax.fori_loop(1, n_ch // _NBUF, body, 0)

        j_last = n_ch - _NBUF
        for b in range(_NBUF):
            gather_wait(j_last + b, b)
            write(j_last + b, b)
        for b in range(_NBUF):
            write_wait(j_last + b, b)

    return run(idx, table_pad)


def _transpose_pad_tc(tT, Vp):
    D, V = tT.shape
    BC = 2048

    def body(x_ref, o_ref):
        o_ref[:, :D] = x_ref[...].T
        o_ref[:, D:] = jnp.zeros((BC, _DP - D), jnp.float32)

    return pl.pallas_call(
        body,
        grid=(pl.cdiv(Vp, BC),),
        in_specs=[pl.BlockSpec((D, BC), lambda i: (0, i))],
        out_specs=pl.BlockSpec((BC, _DP), lambda i: (i, 0)),
        out_shape=jax.ShapeDtypeStruct((Vp, _DP), jnp.float32),
    )(tT)


def kernel(doc_batch, table):
    B, S = doc_batch.shape
    V, D = table.shape
    N = B * S
    n_ch = N // (_NW * _CHUNK)
    idx = doc_batch.astype(jnp.int32).reshape(_NW, n_ch, _CHUNK)
    Vp = V + ((-V) % 8)
    table_pad = _transpose_pad_tc(table.T, Vp)
    out = _gather_call(idx, table_pad, n_ch)
    return out.reshape(B, S, _DP)[:, :, :D]


# speedup vs baseline: 1.7604x; 1.0317x over previous
import functools

import jax
import jax.numpy as jnp
from jax import lax
from jax.experimental import pallas as pl
from jax.experimental.pallas import tpu as pltpu
from jax.experimental.pallas import tpu_sc as plsc

_NUM_CORES = 2
_NUM_SUBCORES = 16
_NW = _NUM_CORES * _NUM_SUBCORES
_CHUNK = 128
_DP = 128
_NBUF = 5


def _gather_call(idx, table_pad, n_ch):
    N = _NW * n_ch * _CHUNK
    mesh = plsc.VectorSubcoreMesh(core_axis_name="c", subcore_axis_name="s")

    @functools.partial(
        pl.kernel,
        mesh=mesh,
        out_type=jax.ShapeDtypeStruct((N, _DP), jnp.float32),
        scratch_types=[
            pltpu.VMEM((n_ch, _CHUNK), jnp.int32),
            pltpu.VMEM((_NBUF, _CHUNK, _DP), jnp.float32),
            pltpu.SemaphoreType.DMA((_NBUF,)),
            pltpu.SemaphoreType.DMA((_NBUF,)),
        ],
    )
    def run(idx_hbm, table_hbm, out_hbm, idx_v, rows_v, gsem, wsem):
        wid = lax.axis_index("s") * _NUM_CORES + lax.axis_index("c")
        base = wid * (n_ch * _CHUNK)
        pltpu.sync_copy(idx_hbm.at[wid], idx_v)

        def gather(j, b):
            pltpu.async_copy(table_hbm.at[idx_v.at[j]], rows_v.at[b], gsem.at[b])

        def gather_wait(j, b):
            pltpu.make_async_copy(
                table_hbm.at[idx_v.at[j]], rows_v.at[b], gsem.at[b]
            ).wait()

        def write(j, b):
            pltpu.async_copy(
                rows_v.at[b],
                out_hbm.at[pl.ds(base + j * _CHUNK, _CHUNK)],
                wsem.at[b],
            )

        def write_wait(j, b):
            pltpu.make_async_copy(
                rows_v.at[b],
                out_hbm.at[pl.ds(base + j * _CHUNK, _CHUNK)],
                wsem.at[b],
            ).wait()

        for b in range(_NBUF):
            gather(b, b)

        def body(g, carry):
            j_prev = (g - 1) * _NBUF
            j_next = g * _NBUF
            for b in range(_NBUF):
                gather_wait(j_prev + b, b)
                write(j_prev + b, b)
            for b in range(_NBUF):
                write_wait(j_prev + b, b)
                gather(j_next + b, b)
            return carry

        lax.fori_loop(1, n_ch // _NBUF, body, 0)

        j_last = n_ch - _NBUF
        for b in range(_NBUF):
            gather_wait(j_last + b, b)
            write(j_last + b, b)
        for b in range(_NBUF):
            write_wait(j_last + b, b)

    return run(idx, table_pad)


def _transpose_pad_tc(tT, Vp):
    D, V = tT.shape
    BC = 8192

    def body(x_ref, o_ref):
        o_ref[:, :D] = x_ref[...].T
        o_ref[:, D:] = jnp.zeros((BC, _DP - D), jnp.float32)

    return pl.pallas_call(
        body,
        grid=(pl.cdiv(Vp, BC),),
        in_specs=[pl.BlockSpec((D, BC), lambda i: (0, i))],
        out_specs=pl.BlockSpec((BC, _DP), lambda i: (i, 0)),
        out_shape=jax.ShapeDtypeStruct((Vp, _DP), jnp.float32),
    )(tT)


def kernel(doc_batch, table):
    B, S = doc_batch.shape
    V, D = table.shape
    N = B * S
    n_ch = N // (_NW * _CHUNK)
    idx = doc_batch.astype(jnp.int32).reshape(_NW, n_ch, _CHUNK)
    Vp = V + ((-V) % 8)
    table_pad = _transpose_pad_tc(table.T, Vp)
    out = _gather_call(idx, table_pad, n_ch)
    return out.reshape(B, S, _DP)[:, :, :D]
